# Initial kernel scaffold; baseline (speedup 1.0000x reference)
import jax
import jax.numpy as jnp
from jax.experimental import pallas as pl


def kernel(next_relations, next_entities, relation_table, entity_table):
    raise NotImplementedError("write your pallas kernel here")


# speedup vs baseline: 2.8198x; 2.8198x over previous
import functools

import jax
import jax.numpy as jnp
from jax import lax
from jax.experimental import pallas as pl
from jax.experimental.pallas import tpu as pltpu
from jax.experimental.pallas import tpu_sc as plsc

EMB = 32
G = 128
CHUNK = 1024
K = CHUNK // G


@functools.lru_cache(maxsize=None)
def _build(n_rows):
    info = plsc.get_sparse_core_info()
    nw = info.num_cores * info.num_subcores
    per_w = n_rows // nw
    nch = per_w // CHUNK
    assert per_w % CHUNK == 0
    mesh = plsc.VectorSubcoreMesh(core_axis_name="c", subcore_axis_name="s")

    @functools.partial(
        pl.kernel,
        mesh=mesh,
        compiler_params=pltpu.CompilerParams(use_tc_tiling_on_sc=False),
        out_type=jax.ShapeDtypeStruct((n_rows, 2, EMB), jnp.float32),
        scratch_types=[
            pltpu.VMEM((K, G), jnp.int32),
            pltpu.VMEM((K, G), jnp.int32),
            pltpu.VMEM((CHUNK, EMB), jnp.float32),
            pltpu.VMEM((CHUNK, EMB), jnp.float32),
            pltpu.SemaphoreType.DMA,
            pltpu.SemaphoreType.DMA,
        ],
    )
    def k(rel_idx_hbm, ent_idx_hbm, rel_tab, ent_tab, out_hbm,
          idxr_v, idxe_v, rowsr_v, rowse_v, sem_r, sem_e):
        wid = lax.axis_index("s") * info.num_cores + lax.axis_index("c")
        base = wid * per_w

        def chunk(g, carry):
            cb = pl.multiple_of(base + g * CHUNK, G)
            row0 = pl.multiple_of(cb // G, 8)
            pltpu.sync_copy(rel_idx_hbm.at[pl.ds(row0, K)], idxr_v)
            pltpu.sync_copy(ent_idx_hbm.at[pl.ds(row0, K)], idxe_v)
            hs = []
            for j in range(K):
                hs.append(pltpu.async_copy(
                    rel_tab.at[idxr_v.at[j]], rowsr_v.at[pl.ds(j * G, G)],
                    sem_r))
                hs.append(pltpu.async_copy(
                    ent_tab.at[idxe_v.at[j]], rowse_v.at[pl.ds(j * G, G)],
                    sem_e))
            for h in hs:
                h.wait()
            pltpu.sync_copy(rowsr_v, out_hbm.at[pl.ds(cb, CHUNK), 0])
            pltpu.sync_copy(rowse_v, out_hbm.at[pl.ds(cb, CHUNK), 1])
            return carry

        lax.fori_loop(0, nch, chunk, 0)

    return k


def kernel(next_relations, next_entities, relation_table, entity_table):
    b, a = next_relations.shape
    n = b * a
    rel_idx = next_relations.reshape(n // G, G).astype(jnp.int32)
    ent_idx = next_entities.reshape(n // G, G).astype(jnp.int32)
    out = _build(n)(rel_idx, ent_idx, relation_table, entity_table)
    return out.reshape(b, a, 2 * EMB)


# speedup vs baseline: 2.8302x; 1.0037x over previous
import functools

import jax
import jax.numpy as jnp
from jax import lax
from jax.experimental import pallas as pl
from jax.experimental.pallas import tpu as pltpu
from jax.experimental.pallas import tpu_sc as plsc

EMB = 32
G = 128
CHUNK = 512
K = CHUNK // G
NBUF = 2


@functools.lru_cache(maxsize=None)
def _build(n_rows):
    info = plsc.get_sparse_core_info()
    nw = info.num_cores * info.num_subcores
    per_w = n_rows // nw
    nch = per_w // CHUNK
    npair = nch // NBUF
    assert per_w % (CHUNK * NBUF) == 0
    mesh = plsc.VectorSubcoreMesh(core_axis_name="c", subcore_axis_name="s")

    @functools.partial(
        pl.kernel,
        mesh=mesh,
        compiler_params=pltpu.CompilerParams(use_tc_tiling_on_sc=False),
        out_type=jax.ShapeDtypeStruct((n_rows, 2, EMB), jnp.float32),
        scratch_types=[
            [pltpu.VMEM((CHUNK,), jnp.int32) for _ in range(NBUF)],
            [pltpu.VMEM((CHUNK,), jnp.int32) for _ in range(NBUF)],
            [pltpu.VMEM((CHUNK, EMB), jnp.float32) for _ in range(NBUF)],
            [pltpu.VMEM((CHUNK, EMB), jnp.float32) for _ in range(NBUF)],
            [pltpu.SemaphoreType.DMA for _ in range(NBUF)],
            [pltpu.SemaphoreType.DMA for _ in range(NBUF)],
            [pltpu.SemaphoreType.DMA for _ in range(NBUF)],
        ],
    )
    def k(rel_idx_hbm, ent_idx_hbm, rel_tab, ent_tab, out_hbm,
          idxr, idxe, rowsr, rowse, semi, semg, semo):
        wid = lax.axis_index("s") * info.num_cores + lax.axis_index("c")
        base = wid * per_w

        def start_idx(g, b):
            cb = pl.multiple_of(base + g * CHUNK, CHUNK)
            pltpu.async_copy(rel_idx_hbm.at[pl.ds(cb, CHUNK)], idxr[b],
                             semi[b])
            pltpu.async_copy(ent_idx_hbm.at[pl.ds(cb, CHUNK)], idxe[b],
                             semi[b])

        def wait_idx(b):
            pltpu.make_async_copy(rel_idx_hbm.at[pl.ds(0, CHUNK)], idxr[b],
                                  semi[b]).wait()
            pltpu.make_async_copy(ent_idx_hbm.at[pl.ds(0, CHUNK)], idxe[b],
                                  semi[b]).wait()

        def start_out(g, b):
            cb = pl.multiple_of(base + g * CHUNK, CHUNK)
            pltpu.async_copy(rowsr[b], out_hbm.at[pl.ds(cb, CHUNK), 0],
                             semo[b])
            pltpu.async_copy(rowse[b], out_hbm.at[pl.ds(cb, CHUNK), 1],
                             semo[b])

        def wait_out(b):
            pltpu.make_async_copy(rowsr[b], out_hbm.at[pl.ds(0, CHUNK), 0],
                                  semo[b]).wait()
            pltpu.make_async_copy(rowse[b], out_hbm.at[pl.ds(0, CHUNK), 1],
                                  semo[b]).wait()

        for b in range(NBUF):
            start_idx(b, b)

        def pair(h, carry):
            for b in range(NBUF):
                g = h * NBUF + b
                wait_idx(b)

                @pl.when(h >= 1)
                def _():
                    wait_out(b)

                hs = []
                for j in range(K):
                    hs.append(pltpu.async_copy(
                        rel_tab.at[idxr[b].at[pl.ds(j * G, G)]],
                        rowsr[b].at[pl.ds(j * G, G)], semg[b]))
                    hs.append(pltpu.async_copy(
                        ent_tab.at[idxe[b].at[pl.ds(j * G, G)]],
                        rowse[b].at[pl.ds(j * G, G)], semg[b]))
                for hd in hs:
                    hd.wait()

                @pl.when(h < npair - 1)
                def _():
                    start_idx(g + NBUF, b)

                start_out(g, b)
            return carry

        lax.fori_loop(0, npair, pair, 0)
        for b in range(NBUF):
            wait_out(b)

    return k


def kernel(next_relations, next_entities, relation_table, entity_table):
    b, a = next_relations.shape
    n = b * a
    rel_idx = next_relations.reshape(n).astype(jnp.int32)
    ent_idx = next_entities.reshape(n).astype(jnp.int32)
    out = _build(n)(rel_idx, ent_idx, relation_table, entity_table)
    return out.reshape(b, a, 2 * EMB)


# speedup vs baseline: 2.8389x; 1.0031x over previous
import functools

import jax
import jax.numpy as jnp
from jax import lax
from jax.experimental import pallas as pl
from jax.experimental.pallas import tpu as pltpu
from jax.experimental.pallas import tpu_sc as plsc

EMB = 32
G = 128
NB = 4
NBUF = 2


@functools.lru_cache(maxsize=None)
def _build(nb_total, na):
    info = plsc.get_sparse_core_info()
    nw = info.num_cores * info.num_subcores
    b_per_w = nb_total // nw
    nch = b_per_w // NB
    rows = NB * na
    assert b_per_w % (NB * NBUF) == 0
    mesh = plsc.VectorSubcoreMesh(core_axis_name="c", subcore_axis_name="s")

    splits = []
    o = 0
    while o < na:
        splits.append((o, min(G, na - o)))
        o += min(G, na - o)

    @functools.partial(
        pl.kernel,
        mesh=mesh,
        compiler_params=pltpu.CompilerParams(use_tc_tiling_on_sc=False),
        out_type=jax.ShapeDtypeStruct((nb_total, na, 2 * EMB), jnp.float32),
        scratch_types=[
            [pltpu.VMEM((rows,), jnp.int32) for _ in range(NBUF)],
            [pltpu.VMEM((rows,), jnp.int32) for _ in range(NBUF)],
            [pltpu.VMEM((NB, na, EMB), jnp.float32) for _ in range(NBUF)],
            [pltpu.VMEM((NB, na, EMB), jnp.float32) for _ in range(NBUF)],
            [pltpu.SemaphoreType.DMA for _ in range(NBUF)],
            [pltpu.SemaphoreType.DMA for _ in range(NBUF)],
            [pltpu.SemaphoreType.DMA for _ in range(NBUF)],
        ],
    )
    def k(rel_idx_hbm, ent_idx_hbm, rel_tab, ent_tab, out_hbm,
          idxr, idxe, rowsr, rowse, semi, semg, semo):
        wid = lax.axis_index("s") * info.num_cores + lax.axis_index("c")
        b_base = wid * b_per_w

        def start_idx(c, b):
            f0 = pl.multiple_of((b_base + c * NB) * na, 8)
            pltpu.async_copy(rel_idx_hbm.at[pl.ds(f0, rows)], idxr[b],
                             semi[b])
            pltpu.async_copy(ent_idx_hbm.at[pl.ds(f0, rows)], idxe[b],
                             semi[b])

        def wait_idx(b):
            pltpu.make_async_copy(rel_idx_hbm.at[pl.ds(0, rows)], idxr[b],
                                  semi[b]).wait()
            pltpu.make_async_copy(ent_idx_hbm.at[pl.ds(0, rows)], idxe[b],
                                  semi[b]).wait()

        def start_out(c, b):
            b0 = pl.multiple_of(b_base + c * NB, NB)
            pltpu.async_copy(rowsr[b], out_hbm.at[pl.ds(b0, NB), :,
                                                  pl.ds(0, EMB)], semo[b])
            pltpu.async_copy(rowse[b], out_hbm.at[pl.ds(b0, NB), :,
                                                  pl.ds(EMB, EMB)], semo[b])

        def wait_out(b):
            pltpu.make_async_copy(rowsr[b], out_hbm.at[pl.ds(0, NB), :,
                                                       pl.ds(0, EMB)],
                                  semo[b]).wait()
            pltpu.make_async_copy(rowse[b], out_hbm.at[pl.ds(0, NB), :,
                                                       pl.ds(EMB, EMB)],
                                  semo[b]).wait()

        for b in range(NBUF):
            start_idx(b, b)

        def pair(h, carry):
            for b in range(NBUF):
                c = h * NBUF + b
                wait_idx(b)

                @pl.when(h >= 1)
                def _():
                    wait_out(b)

                hs = []
                for bb in range(NB):
                    for (o, ln) in splits:
                        hs.append(pltpu.async_copy(
                            rel_tab.at[idxr[b].at[pl.ds(bb * na + o, ln)]],
                            rowsr[b].at[bb, pl.ds(o, ln)], semg[b]))
                        hs.append(pltpu.async_copy(
                            ent_tab.at[idxe[b].at[pl.ds(bb * na + o, ln)]],
                            rowse[b].at[bb, pl.ds(o, ln)], semg[b]))
                for hd in hs:
                    hd.wait()

                @pl.when(h < (nch // NBUF) - 1)
                def _():
                    start_idx(c + NBUF, b)

                start_out(c, b)
            return carry

        lax.fori_loop(0, nch // NBUF, pair, 0)
        for b in range(NBUF):
            wait_out(b)

    return k


def kernel(next_relations, next_entities, relation_table, entity_table):
    b, a = next_relations.shape
    rel_idx = next_relations.reshape(b * a).astype(jnp.int32)
    ent_idx = next_entities.reshape(b * a).astype(jnp.int32)
    return _build(b, a)(rel_idx, ent_idx, relation_table, entity_table)
